# Initial kernel scaffold; baseline (speedup 1.0000x reference)
import jax
import jax.numpy as jnp
from jax.experimental import pallas as pl


def kernel(x, edge_index, w1, b1, w2, b2, w3, b3, cw1, cb1, g1, be1, cw2, cb2, g2, be2, fw, fb, fg, fbe, dw1, db1, dw2, db2, dw3, db3):
    raise NotImplementedError("write your pallas kernel here")


# speedup vs baseline: 17.1438x; 17.1438x over previous
import jax
import jax.numpy as jnp
from jax import lax
from jax.experimental import pallas as pl
from jax.experimental.pallas import tpu as pltpu
from jax.experimental.pallas import tpu_sc as plsc

N = 10000
E = 320000
NC = 2
NS = 16
NW = NC * NS
EPW = E // NW
K = 80
C = EPW // K
SPT = 632
ACCN = NS * SPT


def _sc_agg(F):
    mesh = plsc.VectorSubcoreMesh(core_axis_name="c", subcore_axis_name="s")

    def body(g_hbm, src_hbm, dst_hbm, zero_hbm, out_hbm, sidx, didx, rows, acc, sem):
        c = lax.axis_index("c")
        s = lax.axis_index("s")
        w = c * NS + s
        pltpu.sync_copy(src_hbm.at[w], sidx)
        pltpu.sync_copy(dst_hbm.at[w], didx)
        r0 = s * SPT
        pltpu.sync_copy(zero_hbm, acc.at[pl.ds(r0, SPT)])
        plsc.subcore_barrier()

        def chunk(j, carry):
            pltpu.async_copy(g_hbm.at[sidx.at[j]], rows, sem).wait()
            pltpu.sync_copy(rows, acc.at[didx.at[j]], add=True)
            return carry

        lax.fori_loop(0, C, chunk, 0)
        plsc.subcore_barrier()
        pltpu.sync_copy(acc.at[pl.ds(r0, SPT)], out_hbm.at[c, pl.ds(r0, SPT)])

    return pl.kernel(
        body,
        out_type=jax.ShapeDtypeStruct((NC, ACCN, F), jnp.float32),
        mesh=mesh,
        scratch_types=[
            pltpu.VMEM((C, K), jnp.int32),
            pltpu.VMEM((C, K), jnp.int32),
            pltpu.VMEM((K, F), jnp.float32),
            pltpu.VMEM_SHARED((ACCN, F), jnp.float32),
            pltpu.SemaphoreType.DMA,
        ],
        compiler_params=pltpu.CompilerParams(use_tc_tiling_on_sc=False),
        name=f"sc_agg_f{F}",
    )


def _prep_body(p0, h0p, dinv, g1):
    deg = p0[0, :N, 0:1] + p0[1, :N, 0:1] + 1.0
    dv = lax.rsqrt(deg)
    dinv[...] = dv
    g1[...] = dv * h0p[...]


def _layer_body(p, g, dinv, w, b, gout):
    dv = dinv[...]
    a = dv * (p[0, :N] + p[1, :N] + g[...])
    h = jnp.maximum(jnp.dot(a, w[...], preferred_element_type=jnp.float32, precision=lax.Precision.HIGHEST) + b[...], 0.0)
    gout[...] = dv * h


def _bn(t, gamma, beta, eps=1e-5):
    m = jnp.mean(t, axis=0, keepdims=True)
    v = jnp.mean((t - m) ** 2, axis=0, keepdims=True)
    return gamma * (t - m) * lax.rsqrt(v + eps) + beta


def _tail_body(p, g, dinv, w3, b3, cw1, cb1, gm1, be1, cw2, cb2, gm2, be2,
               fw, fb, fg, fbe, dw1, db1, dw2, db2, dw3, db3, out):
    dv = dinv[...]
    a = dv * (p[0, :N] + p[1, :N] + g[...])
    h = jnp.maximum(jnp.dot(a, w3[...], preferred_element_type=jnp.float32, precision=lax.Precision.HIGHEST) + b3[...], 0.0)
    t = jnp.maximum(_bn(jnp.dot(h, cw1[...], preferred_element_type=jnp.float32, precision=lax.Precision.HIGHEST) + cb1[...],
                        gm1[...], be1[...]), 0.0)
    t = jnp.maximum(_bn(jnp.dot(t, cw2[...], preferred_element_type=jnp.float32, precision=lax.Precision.HIGHEST) + cb2[...],
                        gm2[...], be2[...]), 0.0)
    f = _bn(jnp.dot(t, fw[...], preferred_element_type=jnp.float32, precision=lax.Precision.HIGHEST) + fb[...],
            fg[...], fbe[...])
    z = jnp.max(f, axis=0, keepdims=True)
    o = jnp.maximum(jnp.dot(z, dw1[...], preferred_element_type=jnp.float32, precision=lax.Precision.HIGHEST) + db1[...], 0.0)
    o = jnp.maximum(jnp.dot(o, dw2[...], preferred_element_type=jnp.float32, precision=lax.Precision.HIGHEST) + db2[...], 0.0)
    out[...] = jnp.dot(o, dw3[...], preferred_element_type=jnp.float32, precision=lax.Precision.HIGHEST) + db3[...]


def _tc(fn, out_shape):
    return pl.pallas_call(fn, out_shape=out_shape)


def kernel(x, edge_index, w1, b1, w2, b2, w3, b3, cw1, cb1, g1, be1, cw2, cb2,
           g2, be2, fw, fb, fg, fbe, dw1, db1, dw2, db2, dw3, db3):
    f32 = jnp.float32
    srcw = edge_index[0].reshape(NW, C, K)
    dstw = edge_index[1].reshape(NW, C, K)
    h0p = jnp.pad(x[0].T, ((0, 0), (0, 6)))
    w1p = jnp.pad(w1, ((0, 6), (0, 0)))
    ones_tab = jnp.zeros((N, 8), f32).at[:, 0].set(1.0)
    z8 = jnp.zeros((SPT, 8), f32)
    z64 = jnp.zeros((SPT, 64), f32)
    z128 = jnp.zeros((SPT, 128), f32)

    p0 = _sc_agg(8)(ones_tab, srcw, dstw, z8)
    dinv, gg1 = _tc(_prep_body, (jax.ShapeDtypeStruct((N, 1), f32),
                                 jax.ShapeDtypeStruct((N, 8), f32)))(p0, h0p)

    p1 = _sc_agg(8)(gg1, srcw, dstw, z8)
    gg2 = _tc(_layer_body, jax.ShapeDtypeStruct((N, 64), f32))(
        p1, gg1, dinv, w1p, b1.reshape(1, -1))

    p2 = _sc_agg(64)(gg2, srcw, dstw, z64)
    gg3 = _tc(_layer_body, jax.ShapeDtypeStruct((N, 128), f32))(
        p2, gg2, dinv, w2, b2.reshape(1, -1))

    p3 = _sc_agg(128)(gg3, srcw, dstw, z128)
    o = _tc(_tail_body, jax.ShapeDtypeStruct((1, 4096), f32))(
        p3, gg3, dinv, w3, b3.reshape(1, -1),
        cw1, cb1.reshape(1, -1), g1.reshape(1, -1), be1.reshape(1, -1),
        cw2, cb2.reshape(1, -1), g2.reshape(1, -1), be2.reshape(1, -1),
        fw, fb.reshape(1, -1), fg.reshape(1, -1), fbe.reshape(1, -1),
        dw1, db1.reshape(1, -1), dw2, db2.reshape(1, -1), dw3, db3.reshape(1, -1))
    return o.reshape(-1, 2048, 2)
